# Initial kernel scaffold; baseline (speedup 1.0000x reference)
import jax
import jax.numpy as jnp
from jax.experimental import pallas as pl


def kernel(uv_x, coords, sdf_grid, folding_coords, lw1, lb1, lw2, lb2, lw3, lb3, tw1, tb1, tw2, tb2, tw3, tb3):
    raise NotImplementedError("write your pallas kernel here")


# speedup vs baseline: 12.9083x; 12.9083x over previous
import functools

import jax
import jax.numpy as jnp
import numpy as np
from jax import lax
from jax.experimental import pallas as pl
from jax.experimental.pallas import tpu as pltpu
from jax.experimental.pallas import tpu_sc as plsc

B, N, M, K, D, FEAT, HID = 2, 16384, 2048, 4, 64, 32, 64
NFREQ = 4
PE = 3 * 2 * NFREQ
CIN = FEAT + PE
TW = 48

NC, NS = 2, 16
NW = NC * NS
LANES = 16

TQ = 512
TR = 1024


def _knn_body(c_ref, ft_ref, idx_ref, w_ref):
    q = c_ref[0]
    rt = ft_ref[0]
    qq = jnp.sum(q * q, axis=1, keepdims=True)
    rr = jnp.sum(rt * rt, axis=0, keepdims=True)
    qr = jnp.dot(q, rt, preferred_element_type=jnp.float32)
    neg = 2.0 * qr - qq - rr
    iota = lax.broadcasted_iota(jnp.int32, (TQ, M), 1)
    vals, idxs = [], []
    for k in range(K):
        mk = jnp.max(neg, axis=1, keepdims=True)
        cand = jnp.where(neg == mk, iota, M)
        ik = jnp.min(cand, axis=1, keepdims=True)
        vals.append(mk)
        idxs.append(ik)
        if k < K - 1:
            neg = jnp.where(iota == ik, -jnp.inf, neg)
    dis = jnp.maximum(-jnp.concatenate(vals, axis=1), 0.0)
    w = 1.0 / (jnp.sqrt(dis) + 1e-7)
    w = w / jnp.sum(w, axis=1, keepdims=True)
    b = pl.program_id(0)
    idx_ref[0] = (jnp.concatenate(idxs, axis=1) + b * M).astype(jnp.int32)
    w_ref[0] = w


def _knn(coords, fold_t):
    return pl.pallas_call(
        _knn_body,
        grid=(B, N // TQ),
        in_specs=[
            pl.BlockSpec((1, TQ, 3), lambda b, i: (b, i, 0)),
            pl.BlockSpec((1, 3, M), lambda b, i: (b, 0, 0)),
        ],
        out_specs=[
            pl.BlockSpec((1, TQ, K), lambda b, i: (b, i, 0)),
            pl.BlockSpec((1, TQ, K), lambda b, i: (b, i, 0)),
        ],
        out_shape=[
            jax.ShapeDtypeStruct((B, N, K), jnp.int32),
            jax.ShapeDtypeStruct((B, N, K), jnp.float32),
        ],
    )(coords, fold_t)


TOT = K * B * N
PER_W = TOT // NW
GC = 128
NCHUNK = PER_W // GC
GFIRE = 4


def _gather_body(table_hbm, idx_hbm, out_hbm, *scratch):
    idx_vs = scratch[:GFIRE]
    rows_vs = scratch[GFIRE:2 * GFIRE]
    sems = scratch[2 * GFIRE:]
    wid = lax.axis_index("s") * NC + lax.axis_index("c")
    base = wid * PER_W

    def issue(c, slot):
        off = base + c * GC
        pltpu.sync_copy(idx_hbm.at[pl.ds(off, GC)], idx_vs[slot])
        pltpu.async_copy(table_hbm.at[idx_vs[slot]], rows_vs[slot],
                         sems[slot])

    def drain(c, slot):
        off = base + c * GC
        pltpu.make_async_copy(table_hbm.at[idx_vs[slot]], rows_vs[slot],
                              sems[slot]).wait()
        pltpu.sync_copy(rows_vs[slot], out_hbm.at[pl.ds(off, GC)])

    for g in range(NCHUNK // GFIRE):
        for s in range(GFIRE):
            issue(g * GFIRE + s, s)
        for s in range(GFIRE):
            drain(g * GFIRE + s, s)


def _sc_gather(table, idx_flat):
    mesh = plsc.VectorSubcoreMesh(core_axis_name="c", subcore_axis_name="s",
                                  num_cores=NC, num_subcores=NS)
    return pl.kernel(
        _gather_body,
        out_type=jax.ShapeDtypeStruct((TOT, TW), jnp.float32),
        mesh=mesh,
        compiler_params=pltpu.CompilerParams(use_tc_tiling_on_sc=False),
        scratch_types=(
            [pltpu.VMEM((GC,), jnp.int32) for _ in range(GFIRE)]
            + [pltpu.VMEM((GC, TW), jnp.float32) for _ in range(GFIRE)]
            + [pltpu.SemaphoreType.DMA for _ in range(GFIRE)]
        ),
    )(table, idx_flat)


TOTP = B * N
PPW = TOTP // NW
GB = 8
PGRP = GB * LANES
ROWS = 4 * PGRP
NPCH = PPW // PGRP


def _floor_f(x):
    t = x.astype(jnp.int32)
    tf = t.astype(jnp.float32)
    return jnp.where(tf > x, t - 1, t)


def _sdf_body(vol_hbm, cx_hbm, cy_hbm, cz_hbm, sig_hbm, *scratch):
    cx_v, cy_v, cz_v = scratch[0:3]
    cidx = scratch[3:3 + 8]
    vals = scratch[11:11 + 8]
    out_v = scratch[19]
    sem = scratch[20]
    wid = lax.axis_index("s") * NC + lax.axis_index("c")
    base = wid * PPW
    boff3 = (base // N) * (D * D * D)

    pltpu.sync_copy(cx_hbm.at[pl.ds(base, PPW)], cx_v)
    pltpu.sync_copy(cy_hbm.at[pl.ds(base, PPW)], cy_v)
    pltpu.sync_copy(cz_hbm.at[pl.ds(base, PPW)], cz_v)

    def grid_coord(cv, off16):
        c = cv[pl.ds(off16, LANES)]
        x = ((c * 2.0 + 1.0) * D - 1.0) * 0.5
        x0 = _floor_f(x)
        w = x - x0.astype(jnp.float32)
        v0 = x0 >= 0
        v1 = x0 + 1 <= D - 1
        c0 = jnp.clip(x0, 0, D - 1)
        c1 = jnp.minimum(x0 + 1, D - 1)
        return w, v0, v1, c0, c1

    def chunk(ci, carry):
        p0 = ci * PGRP
        for g in range(GB):
            off16 = p0 + g * LANES
            _, _, _, x0c, x1c = grid_coord(cx_v, off16)
            _, _, _, y0c, y1c = grid_coord(cy_v, off16)
            _, _, _, z0c, z1c = grid_coord(cz_v, off16)
            for c in range(8):
                dz, dy, dx = (c >> 2) & 1, (c >> 1) & 1, c & 1
                zc = z1c if dz else z0c
                yc = y1c if dy else y0c
                xc = x1c if dx else x0c
                cidx[c][pl.ds(g * LANES, LANES)] = ((zc * D + yc) * D + xc
                                                    + boff3)
        for c in range(8):
            pltpu.async_copy(vol_hbm.at[cidx[c]], vals[c], sem)
        for c in range(8):
            pltpu.make_async_copy(vol_hbm.at[cidx[c]], vals[c], sem).wait()

        for g in range(GB):
            off16 = p0 + g * LANES
            wx, vx0, vx1, _, _ = grid_coord(cx_v, off16)
            wy, vy0, vy1, _, _ = grid_coord(cy_v, off16)
            wz, vz0, vz1, _, _ = grid_coord(cz_v, off16)
            acc = jnp.zeros((LANES,), jnp.float32)
            for c in range(8):
                dz, dy, dx = (c >> 2) & 1, (c >> 1) & 1, c & 1
                v = vals[c][pl.ds(g * LANES, LANES)]
                wc = ((wz if dz else 1.0 - wz) * (wy if dy else 1.0 - wy)
                      * (wx if dx else 1.0 - wx))
                ok = ((vz1 if dz else vz0) & (vy1 if dy else vy0)
                      & (vx1 if dx else vx0))
                acc = acc + wc * jnp.where(ok, v - 100.0, 0.0)
            sdf = acc + 100.0
            e = jnp.exp(-jnp.abs(sdf) * (1.0 / 0.005)) - 1.0
            sg = jnp.where(sdf > 0, 1.0, jnp.where(sdf < 0, -1.0, 0.0))
            sig = (1.0 / 0.005) * (0.5 + 0.5 * sg * e)
            out_v[pl.ds(p0 + g * LANES, LANES)] = sig
        return carry

    lax.fori_loop(0, NPCH, chunk, 0)
    pltpu.sync_copy(out_v, sig_hbm.at[pl.ds(base, PPW)])


def _sc_sdf(vol, cx, cy, cz):
    mesh = plsc.VectorSubcoreMesh(core_axis_name="c", subcore_axis_name="s",
                                  num_cores=NC, num_subcores=NS)
    return pl.kernel(
        _sdf_body,
        out_type=jax.ShapeDtypeStruct((TOTP,), jnp.float32),
        mesh=mesh,
        scratch_types=(
            [pltpu.VMEM((PPW,), jnp.float32) for _ in range(3)]
            + [pltpu.VMEM((PGRP,), jnp.int32) for _ in range(8)]
            + [pltpu.VMEM((PGRP,), jnp.float32) for _ in range(8)]
            + [pltpu.VMEM((PPW,), jnp.float32), pltpu.SemaphoreType.DMA]
        ),
    )(vol, cx, cy, cz)


def _lrelu(y):
    return jnp.where(y >= 0, y, 0.2 * y) * np.sqrt(2.0).astype(np.float32)


def _mlp_body(g_ref, c_ref, w_ref, s_ref, lw1t_ref, lb1_ref, lw2t_ref,
              lb2_ref, lw3t_ref, lb3_ref, tw1t_ref, tb1_ref, tw2t_ref,
              tb2_ref, tw3t_ref, tb3_ref, out_ref):
    fi = lax.broadcasted_iota(jnp.int32, (1, NFREQ), 1)
    freqs = (jnp.int32(1) << fi).astype(jnp.float32) * np.float32(np.pi)
    w1 = lw1t_ref[:] * np.float32(1.0 / np.sqrt(CIN))
    w2 = lw2t_ref[:] * np.float32(1.0 / np.sqrt(HID))
    w3 = lw3t_ref[:] * np.float32(1.0 / np.sqrt(HID))
    t1 = tw1t_ref[:] * np.float32(1.0 / np.sqrt(FEAT))
    t2 = tw2t_ref[:] * np.float32(1.0 / np.sqrt(HID))
    t3 = tw3t_ref[:] * np.float32(1.0 / np.sqrt(HID))
    c = c_ref[:]
    acc = jnp.zeros((TR, FEAT), jnp.float32)
    for k in range(K):
        x = g_ref[k]
        feats = x[:, :FEAT]
        local = c - x[:, FEAT:FEAT + 3]
        parts = [feats]
        for d in range(3):
            xf = local[:, d:d + 1] * freqs
            parts.append(jnp.sin(xf))
            parts.append(jnp.cos(xf))
        h = jnp.concatenate(parts, axis=1)
        h = _lrelu(jnp.dot(h, w1, preferred_element_type=jnp.float32)
                   + lb1_ref[:])
        h = _lrelu(jnp.dot(h, w2, preferred_element_type=jnp.float32)
                   + lb2_ref[:])
        h = (jnp.dot(h, w3, preferred_element_type=jnp.float32)
             + lb3_ref[:])
        acc = acc + h * w_ref[:, k:k + 1]
    t = _lrelu(jnp.dot(acc, t1, preferred_element_type=jnp.float32)
               + tb1_ref[:])
    t = _lrelu(jnp.dot(t, t2, preferred_element_type=jnp.float32)
               + tb2_ref[:])
    rgb = (jnp.dot(t, t3, preferred_element_type=jnp.float32)
           + tb3_ref[:])
    out_ref[:] = jnp.concatenate([rgb, s_ref[:]], axis=1)


def _mlp(g, coords_f, wts, sig, lw1t, lb1, lw2t, lb2, lw3t, lb3,
         tw1t, tb1, tw2t, tb2, tw3t, tb3):
    nrow = B * N
    full = lambda shape: pl.BlockSpec(shape, lambda i: tuple(0 for _ in shape))
    return pl.pallas_call(
        _mlp_body,
        grid=(nrow // TR,),
        in_specs=[
            pl.BlockSpec((K, TR, TW), lambda i: (0, i, 0)),
            pl.BlockSpec((TR, 3), lambda i: (i, 0)),
            pl.BlockSpec((TR, K), lambda i: (i, 0)),
            pl.BlockSpec((TR, 1), lambda i: (i, 0)),
            full((CIN, HID)), full((1, HID)),
            full((HID, HID)), full((1, HID)),
            full((HID, FEAT)), full((1, FEAT)),
            full((FEAT, HID)), full((1, HID)),
            full((HID, HID)), full((1, HID)),
            full((HID, 3)), full((1, 3)),
        ],
        out_specs=pl.BlockSpec((TR, 4), lambda i: (i, 0)),
        out_shape=jax.ShapeDtypeStruct((nrow, 4), jnp.float32),
    )(g, coords_f, wts, sig, lw1t, lb1, lw2t, lb2, lw3t, lb3,
      tw1t, tb1, tw2t, tb2, tw3t, tb3)


def kernel(uv_x, coords, sdf_grid, folding_coords, lw1, lb1, lw2, lb2, lw3,
           lb3, tw1, tb1, tw2, tb2, tw3, tb3):
    fold_t = folding_coords.transpose(0, 2, 1)
    idx_g, wts = _knn(coords, fold_t)

    pad = jnp.zeros((B, M, TW - FEAT - 3), jnp.float32)
    table = jnp.concatenate([uv_x, folding_coords, pad], axis=-1)
    table = table.reshape(B * M, TW)
    idx_flat = idx_g.reshape(B * N, K).T.reshape(TOT)
    g = _sc_gather(table, idx_flat).reshape(K, B * N, TW)

    vol = sdf_grid.reshape(B * D * D * D)
    cf = coords.reshape(B * N, 3)
    sig = _sc_sdf(vol, cf[:, 0], cf[:, 1], cf[:, 2])

    out = _mlp(g, coords.reshape(B * N, 3), wts.reshape(B * N, K),
               sig.reshape(B * N, 1),
               lw1.T, lb1.reshape(1, HID), lw2.T, lb2.reshape(1, HID),
               lw3.T, lb3.reshape(1, FEAT), tw1.T, tb1.reshape(1, HID),
               tw2.T, tb2.reshape(1, HID), tw3.T, tb3.reshape(1, 3))
    return out.reshape(B, N, 4)


# speedup vs baseline: 32.0717x; 2.4846x over previous
import functools

import jax
import jax.numpy as jnp
import numpy as np
from jax import lax
from jax.experimental import pallas as pl
from jax.experimental.pallas import tpu as pltpu
from jax.experimental.pallas import tpu_sc as plsc

B, N, M, K, D, FEAT, HID = 2, 16384, 2048, 4, 64, 32, 64
NFREQ = 4
PE = 3 * 2 * NFREQ
CIN = FEAT + PE
TW = 48

NC, NS = 2, 16
NW = NC * NS
LANES = 16

TQ = 512
TR = 1024


def _knn_body(c_ref, ft_ref, idx_ref, w_ref):
    q = c_ref[0]
    rt = ft_ref[0]
    qq = jnp.sum(q * q, axis=1, keepdims=True)
    rr = jnp.sum(rt * rt, axis=0, keepdims=True)
    qr = jnp.dot(q, rt, preferred_element_type=jnp.float32)
    neg = 2.0 * qr - qq - rr
    iota = lax.broadcasted_iota(jnp.int32, (TQ, M), 1)
    vals, idxs = [], []
    for k in range(K):
        mk = jnp.max(neg, axis=1, keepdims=True)
        cand = jnp.where(neg == mk, iota, M)
        ik = jnp.min(cand, axis=1, keepdims=True)
        vals.append(mk)
        idxs.append(ik)
        if k < K - 1:
            neg = jnp.where(iota == ik, -jnp.inf, neg)
    dis = jnp.maximum(-jnp.concatenate(vals, axis=1), 0.0)
    w = 1.0 / (jnp.sqrt(dis) + 1e-7)
    w = w / jnp.sum(w, axis=1, keepdims=True)
    b = pl.program_id(0)
    idx_ref[0] = (jnp.concatenate(idxs, axis=1) + b * M).astype(jnp.int32)
    w_ref[0] = w


def _knn(coords, fold_t):
    return pl.pallas_call(
        _knn_body,
        grid=(B, N // TQ),
        in_specs=[
            pl.BlockSpec((1, TQ, 3), lambda b, i: (b, i, 0)),
            pl.BlockSpec((1, 3, M), lambda b, i: (b, 0, 0)),
        ],
        out_specs=[
            pl.BlockSpec((1, TQ, K), lambda b, i: (b, i, 0)),
            pl.BlockSpec((1, TQ, K), lambda b, i: (b, i, 0)),
        ],
        out_shape=[
            jax.ShapeDtypeStruct((B, N, K), jnp.int32),
            jax.ShapeDtypeStruct((B, N, K), jnp.float32),
        ],
    )(coords, fold_t)


TOT = K * B * N
PER_W = TOT // NW
GC = 128
NCHUNK = PER_W // GC
GFIRE = 4


def _gather_body(table_hbm, idx_hbm, out_hbm, *scratch):
    idx_vs = scratch[:GFIRE]
    rows_vs = scratch[GFIRE:2 * GFIRE]
    sems = scratch[2 * GFIRE:]
    wid = lax.axis_index("s") * NC + lax.axis_index("c")
    base = wid * PER_W

    def issue(c, slot):
        off = base + c * GC
        pltpu.sync_copy(idx_hbm.at[pl.ds(off, GC)], idx_vs[slot])
        pltpu.async_copy(table_hbm.at[idx_vs[slot]], rows_vs[slot],
                         sems[slot])

    def drain(c, slot):
        off = base + c * GC
        pltpu.make_async_copy(table_hbm.at[idx_vs[slot]], rows_vs[slot],
                              sems[slot]).wait()
        pltpu.sync_copy(rows_vs[slot], out_hbm.at[pl.ds(off, GC)])

    for g in range(NCHUNK // GFIRE):
        for s in range(GFIRE):
            issue(g * GFIRE + s, s)
        for s in range(GFIRE):
            drain(g * GFIRE + s, s)


def _sc_gather(table, idx_flat):
    mesh = plsc.VectorSubcoreMesh(core_axis_name="c", subcore_axis_name="s",
                                  num_cores=NC, num_subcores=NS)
    return pl.kernel(
        _gather_body,
        out_type=jax.ShapeDtypeStruct((TOT, TW), jnp.float32),
        mesh=mesh,
        compiler_params=pltpu.CompilerParams(use_tc_tiling_on_sc=False),
        scratch_types=(
            [pltpu.VMEM((GC,), jnp.int32) for _ in range(GFIRE)]
            + [pltpu.VMEM((GC, TW), jnp.float32) for _ in range(GFIRE)]
            + [pltpu.SemaphoreType.DMA for _ in range(GFIRE)]
        ),
    )(table, idx_flat)


TOTP = B * N
PPW = TOTP // NW
GB = 8
PGRP = GB * LANES
ROWS = 4 * PGRP
NPCH = PPW // PGRP


def _floor_f(x):
    t = x.astype(jnp.int32)
    tf = t.astype(jnp.float32)
    return jnp.where(tf > x, t - 1, t)


def _sdf_body(vol_hbm, cx_hbm, cy_hbm, cz_hbm, sig_hbm, *scratch):
    cx_v, cy_v, cz_v = scratch[0:3]
    cidx = scratch[3:3 + 8]
    vals = scratch[11:11 + 8]
    out_v = scratch[19]
    sem = scratch[20]
    wid = lax.axis_index("s") * NC + lax.axis_index("c")
    base = wid * PPW
    boff3 = (base // N) * (D * D * D)

    pltpu.sync_copy(cx_hbm.at[pl.ds(base, PPW)], cx_v)
    pltpu.sync_copy(cy_hbm.at[pl.ds(base, PPW)], cy_v)
    pltpu.sync_copy(cz_hbm.at[pl.ds(base, PPW)], cz_v)

    def grid_coord(cv, off16):
        c = cv[pl.ds(off16, LANES)]
        x = ((c * 2.0 + 1.0) * D - 1.0) * 0.5
        x0 = _floor_f(x)
        w = x - x0.astype(jnp.float32)
        v0 = x0 >= 0
        v1 = x0 + 1 <= D - 1
        c0 = jnp.clip(x0, 0, D - 1)
        c1 = jnp.minimum(x0 + 1, D - 1)
        return w, v0, v1, c0, c1

    def chunk(ci, carry):
        p0 = ci * PGRP
        for g in range(GB):
            off16 = p0 + g * LANES
            _, _, _, x0c, x1c = grid_coord(cx_v, off16)
            _, _, _, y0c, y1c = grid_coord(cy_v, off16)
            _, _, _, z0c, z1c = grid_coord(cz_v, off16)
            for c in range(8):
                dz, dy, dx = (c >> 2) & 1, (c >> 1) & 1, c & 1
                zc = z1c if dz else z0c
                yc = y1c if dy else y0c
                xc = x1c if dx else x0c
                cidx[c][pl.ds(g * LANES, LANES)] = ((zc * D + yc) * D + xc
                                                    + boff3)
        for c in range(8):
            pltpu.async_copy(vol_hbm.at[cidx[c]], vals[c], sem)
        for c in range(8):
            pltpu.make_async_copy(vol_hbm.at[cidx[c]], vals[c], sem).wait()

        for g in range(GB):
            off16 = p0 + g * LANES
            wx, vx0, vx1, _, _ = grid_coord(cx_v, off16)
            wy, vy0, vy1, _, _ = grid_coord(cy_v, off16)
            wz, vz0, vz1, _, _ = grid_coord(cz_v, off16)
            acc = jnp.zeros((LANES,), jnp.float32)
            for c in range(8):
                dz, dy, dx = (c >> 2) & 1, (c >> 1) & 1, c & 1
                v = vals[c][pl.ds(g * LANES, LANES)]
                wc = ((wz if dz else 1.0 - wz) * (wy if dy else 1.0 - wy)
                      * (wx if dx else 1.0 - wx))
                ok = ((vz1 if dz else vz0) & (vy1 if dy else vy0)
                      & (vx1 if dx else vx0))
                acc = acc + wc * jnp.where(ok, v - 100.0, 0.0)
            sdf = acc + 100.0
            e = jnp.exp(-jnp.abs(sdf) * (1.0 / 0.005)) - 1.0
            sg = jnp.where(sdf > 0, 1.0, jnp.where(sdf < 0, -1.0, 0.0))
            sig = (1.0 / 0.005) * (0.5 + 0.5 * sg * e)
            out_v[pl.ds(p0 + g * LANES, LANES)] = sig
        return carry

    lax.fori_loop(0, NPCH, chunk, 0)
    pltpu.sync_copy(out_v, sig_hbm.at[pl.ds(base, PPW)])


def _sc_sdf(vol, cx, cy, cz):
    mesh = plsc.VectorSubcoreMesh(core_axis_name="c", subcore_axis_name="s",
                                  num_cores=NC, num_subcores=NS)
    return pl.kernel(
        _sdf_body,
        out_type=jax.ShapeDtypeStruct((TOTP,), jnp.float32),
        mesh=mesh,
        scratch_types=(
            [pltpu.VMEM((PPW,), jnp.float32) for _ in range(3)]
            + [pltpu.VMEM((PGRP,), jnp.int32) for _ in range(8)]
            + [pltpu.VMEM((PGRP,), jnp.float32) for _ in range(8)]
            + [pltpu.VMEM((PPW,), jnp.float32), pltpu.SemaphoreType.DMA]
        ),
    )(vol, cx, cy, cz)


def _lrelu(y):
    return jnp.where(y >= 0, y, 0.2 * y) * np.sqrt(2.0).astype(np.float32)


def _mlp_body(g_ref, ct_ref, w_ref, s_ref, lw1_ref, lb1_ref, lw2_ref,
              lb2_ref, lw3_ref, lb3_ref, tw1_ref, tb1_ref, tw2_ref,
              tb2_ref, tw3_ref, tb3_ref, out_ref):
    w1 = lw1_ref[:] * np.float32(1.0 / np.sqrt(CIN))
    w2 = lw2_ref[:] * np.float32(1.0 / np.sqrt(HID))
    w3 = lw3_ref[:] * np.float32(1.0 / np.sqrt(HID))
    t1 = tw1_ref[:] * np.float32(1.0 / np.sqrt(FEAT))
    t2 = tw2_ref[:] * np.float32(1.0 / np.sqrt(HID))
    t3 = tw3_ref[:] * np.float32(1.0 / np.sqrt(HID))
    x_all = g_ref[:].reshape(K * TR, TW)
    xT = x_all.T
    cT = ct_ref[:]
    cT_all = jnp.concatenate([cT] * K, axis=1)
    localT = cT_all - xT[FEAT:FEAT + 3]
    ri = lax.broadcasted_iota(jnp.int32, (2 * NFREQ, 1), 0)
    fcol = ((jnp.int32(1) << (ri & (NFREQ - 1))).astype(jnp.float32)
            * np.float32(np.pi))
    ocol = jnp.where(ri >= NFREQ, np.float32(np.pi / 2), np.float32(0.0))
    parts = []
    for d in range(3):
        row8 = jnp.concatenate([localT[d:d + 1]] * (2 * NFREQ), axis=0)
        parts.append(row8 * fcol + ocol)
    peT = jnp.sin(jnp.concatenate(parts, axis=0))
    hT = jnp.concatenate([xT[:FEAT], peT], axis=0)
    hT = _lrelu(jnp.dot(w1, hT, preferred_element_type=jnp.float32)
                + lb1_ref[:])
    hT = _lrelu(jnp.dot(w2, hT, preferred_element_type=jnp.float32)
                + lb2_ref[:])
    hT = (jnp.dot(w3, hT, preferred_element_type=jnp.float32)
          + lb3_ref[:])
    accT = jnp.zeros((FEAT, TR), jnp.float32)
    for k in range(K):
        accT = accT + hT[:, k * TR:(k + 1) * TR] * w_ref[k:k + 1, :]
    tT = _lrelu(jnp.dot(t1, accT, preferred_element_type=jnp.float32)
                + tb1_ref[:])
    tT = _lrelu(jnp.dot(t2, tT, preferred_element_type=jnp.float32)
                + tb2_ref[:])
    rgbT = (jnp.dot(t3, tT, preferred_element_type=jnp.float32)
            + tb3_ref[:])
    out_ref[:] = jnp.concatenate([rgbT, s_ref[:]], axis=0)


def _mlp(g, coords_t, wts_t, sig_row, lw1, lb1, lw2, lb2, lw3, lb3,
         tw1, tb1, tw2, tb2, tw3, tb3):
    nrow = B * N
    full = lambda shape: pl.BlockSpec(shape, lambda i: tuple(0 for _ in shape))
    return pl.pallas_call(
        _mlp_body,
        grid=(nrow // TR,),
        in_specs=[
            pl.BlockSpec((K, TR, TW), lambda i: (0, i, 0)),
            pl.BlockSpec((3, TR), lambda i: (0, i)),
            pl.BlockSpec((K, TR), lambda i: (0, i)),
            pl.BlockSpec((1, TR), lambda i: (0, i)),
            full((HID, CIN)), full((HID, 1)),
            full((HID, HID)), full((HID, 1)),
            full((FEAT, HID)), full((FEAT, 1)),
            full((HID, FEAT)), full((HID, 1)),
            full((HID, HID)), full((HID, 1)),
            full((3, HID)), full((3, 1)),
        ],
        out_specs=pl.BlockSpec((4, TR), lambda i: (0, i)),
        out_shape=jax.ShapeDtypeStruct((4, nrow), jnp.float32),
    )(g, coords_t, wts_t, sig_row, lw1, lb1, lw2, lb2, lw3, lb3,
      tw1, tb1, tw2, tb2, tw3, tb3)


def kernel(uv_x, coords, sdf_grid, folding_coords, lw1, lb1, lw2, lb2, lw3,
           lb3, tw1, tb1, tw2, tb2, tw3, tb3):
    fold_t = folding_coords.transpose(0, 2, 1)
    idx_g, wts = _knn(coords, fold_t)

    pad = jnp.zeros((B, M, TW - FEAT - 3), jnp.float32)
    table = jnp.concatenate([uv_x, folding_coords, pad], axis=-1)
    table = table.reshape(B * M, TW)
    idx_flat = idx_g.reshape(B * N, K).T.reshape(TOT)
    g = _sc_gather(table, idx_flat).reshape(K, B * N, TW)

    vol = sdf_grid.reshape(B * D * D * D)
    cf = coords.reshape(B * N, 3)
    sig = _sc_sdf(vol, cf[:, 0], cf[:, 1], cf[:, 2])

    out = _mlp(g, coords.reshape(B * N, 3).T, wts.reshape(B * N, K).T,
               sig.reshape(1, B * N),
               lw1, lb1.reshape(HID, 1), lw2, lb2.reshape(HID, 1),
               lw3, lb3.reshape(FEAT, 1), tw1, tb1.reshape(HID, 1),
               tw2, tb2.reshape(HID, 1), tw3, tb3.reshape(3, 1))
    return out.T.reshape(B, N, 4)
